# baseline (device time: 169650 ns/iter reference)
import jax
import jax.numpy as jnp
from jax import lax
from jax.experimental import pallas as pl
from jax.experimental.pallas import tpu as pltpu

N_DEV = 8
SQ = 2048
SKV = 2048
D_MODEL = 1024
DH = 128
H_LOCAL = 8
WIN = 128
QBLK = 256
KSPAN = 512
SCALE = 0.08838834764831843

_GROUPS = (
    (0, 768, (0, 1, 2)),
    (768, 640, (1, 2, 0)),
    (1408, 640, (2, 0, 1)),
)
_COMM_OFF = []
_off = 0
for _base, _rows, _order in _GROUPS:
    _offs = []
    for _s in range(3):
        _offs.append(_off)
        _off += _rows >> (_s + 1)
    _COMM_OFF.append(tuple(_offs))
_COMM_ROWS = _off

_NO_COMM = False


def _body(
    x_ref, wq_ref, k_ref, v_ref, wo_ref, out_ref,
    comm_ref, qbf_scr, ctx_scr, kbf_scr, vbf_scr,
    rs_send, rs_recv, ag_send, ag_recv, ready_sem,
):
    s = pl.program_id(0)
    pos = lax.axis_index("i")
    rows = s * QBLK

    @pl.when(s == 0)
    def _():
        kbf_scr[...] = k_ref[...].astype(jnp.bfloat16)
        vbf_scr[...] = v_ref[...].astype(jnp.bfloat16)
    start = jnp.clip(s * 2 - 1, 0, (SKV - KSPAN) // 128) * 128

    qbf_scr[...] = (
        jnp.dot(
            x_ref[pl.ds(rows, QBLK), :],
            wq_ref[...],
            preferred_element_type=jnp.float32,
        )
        * SCALE
    ).astype(jnp.bfloat16)

    qi = rows + lax.broadcasted_iota(jnp.int32, (QBLK, KSPAN), 0)
    ki = start + lax.broadcasted_iota(jnp.int32, (QBLK, KSPAN), 1)
    win_mask = jnp.abs(qi - ki) <= WIN

    def head(h, carry):
        qh = qbf_scr[:, pl.ds(h * DH, DH)]
        kblk = kbf_scr[pl.ds(start, KSPAN), pl.ds(h * DH, DH)]
        sc = lax.dot_general(
            qh, kblk, (((1,), (1,)), ((), ())),
            preferred_element_type=jnp.float32,
        )
        w = jnp.where(win_mask, jnp.exp(sc), 0.0)
        denom = jnp.sum(w, axis=1, keepdims=True)
        vblk = vbf_scr[pl.ds(start, KSPAN), pl.ds(h * DH, DH)]
        ctx = (
            jnp.dot(
                w.astype(jnp.bfloat16), vblk,
                preferred_element_type=jnp.float32,
            )
            / denom
        )
        ctx_scr[:, pl.ds(h * DH, DH)] = ctx.astype(jnp.bfloat16)
        return carry

    lax.fori_loop(0, H_LOCAL, head, 0)

    out_ref[pl.ds(rows, QBLK), :] = jnp.dot(
        ctx_scr[...], wo_ref[...], preferred_element_type=jnp.float32
    )

    if _NO_COMM:
        return

    q4 = lax.rem(pos, 4)
    zb = pos // 4
    xb = lax.rem((q4 + 1) // 2, 2)
    yb = q4 // 2
    partners = [
        zb * 4 + jnp.bitwise_xor(q4, 1),
        zb * 4 + (3 - q4),
        lax.rem(pos + 4, N_DEV),
    ]
    bits = [xb, yb, zb]

    def rs_stage0_desc(gi):
        base, grows, order = _GROUPS[gi]
        size = grows >> 1
        b = bits[order[0]]
        return pltpu.make_async_remote_copy(
            src_ref=out_ref.at[pl.ds(base + (1 - b) * size, size), :],
            dst_ref=comm_ref.at[pl.ds(_COMM_OFF[gi][0], size), :],
            send_sem=rs_send.at[gi, 0],
            recv_sem=rs_recv.at[gi, 0],
            device_id=(partners[order[0]],),
            device_id_type=pl.DeviceIdType.MESH,
        )

    @pl.when(s == N_DEV - 2)
    def _():
        rs_stage0_desc(0).start()
        rs_stage0_desc(1).start()

    @pl.when(s == N_DEV - 1)
    def _():
        rs_stage0_desc(2).start()
        keep = []
        for gi, (base, grows, order) in enumerate(_GROUPS):
            size = grows >> 1
            keep.append(base + bits[order[0]] * size)
            rdma = rs_stage0_desc(gi)
            rdma.wait_recv()
            out_ref[pl.ds(keep[gi], size), :] += comm_ref[
                pl.ds(_COMM_OFF[gi][0], size), :
            ]
            rdma.wait_send()
        for st in (1, 2):
            rdmas = []
            for gi, (base, grows, order) in enumerate(_GROUPS):
                size = grows >> (st + 1)
                d = order[st]
                b = bits[d]
                send_start = keep[gi] + (1 - b) * size
                keep[gi] = keep[gi] + b * size
                rdma = pltpu.make_async_remote_copy(
                    src_ref=out_ref.at[pl.ds(send_start, size), :],
                    dst_ref=comm_ref.at[pl.ds(_COMM_OFF[gi][st], size), :],
                    send_sem=rs_send.at[gi, st],
                    recv_sem=rs_recv.at[gi, st],
                    device_id=(partners[d],),
                    device_id_type=pl.DeviceIdType.MESH,
                )
                rdma.start()
                rdmas.append(rdma)
            for gi, (base, grows, order) in enumerate(_GROUPS):
                size = grows >> (st + 1)
                rdmas[gi].wait_recv()
                out_ref[pl.ds(keep[gi], size), :] += comm_ref[
                    pl.ds(_COMM_OFF[gi][st], size), :
                ]
                rdmas[gi].wait_send()

        for d in range(3):
            pl.semaphore_signal(
                ready_sem.at[d],
                inc=1,
                device_id=(partners[d],),
                device_id_type=pl.DeviceIdType.MESH,
            )
        for d in range(3):
            pl.semaphore_wait(ready_sem.at[d], 1)

        cur = keep
        for st in (2, 1, 0):
            rdmas = []
            for gi, (base, grows, order) in enumerate(_GROUPS):
                size = grows >> (st + 1)
                rdma = pltpu.make_async_remote_copy(
                    src_ref=out_ref.at[pl.ds(cur[gi], size), :],
                    dst_ref=out_ref.at[pl.ds(cur[gi], size), :],
                    send_sem=ag_send.at[gi, st],
                    recv_sem=ag_recv.at[gi, st],
                    device_id=(partners[order[st]],),
                    device_id_type=pl.DeviceIdType.MESH,
                )
                rdma.start()
                rdmas.append(rdma)
            for gi, (base, grows, order) in enumerate(_GROUPS):
                size = grows >> (st + 1)
                rdmas[gi].wait_recv()
                rdmas[gi].wait_send()
                cur[gi] = cur[gi] - bits[order[st]] * size


def kernel(x, Wq, K_ext, V_ext, Wo):
    pos = lax.axis_index("i")
    K = lax.dynamic_slice_in_dim(
        K_ext[0], pos * H_LOCAL, H_LOCAL, axis=1
    ).reshape(SKV, H_LOCAL * DH)
    V = lax.dynamic_slice_in_dim(
        V_ext[0], pos * H_LOCAL, H_LOCAL, axis=1
    ).reshape(SKV, H_LOCAL * DH)

    out = pl.pallas_call(
        _body,
        grid=(N_DEV,),
        in_specs=[
            pl.BlockSpec((SQ, D_MODEL), lambda s: (0, 0)),
            pl.BlockSpec((D_MODEL, H_LOCAL * DH), lambda s: (0, 0)),
            pl.BlockSpec((SKV, H_LOCAL * DH), lambda s: (0, 0)),
            pl.BlockSpec((SKV, H_LOCAL * DH), lambda s: (0, 0)),
            pl.BlockSpec((H_LOCAL * DH, D_MODEL), lambda s: (0, 0)),
        ],
        out_specs=pl.BlockSpec((SQ, D_MODEL), lambda s: (0, 0)),
        out_shape=jax.ShapeDtypeStruct((SQ, D_MODEL), jnp.float32),
        scratch_shapes=[
            pltpu.VMEM((_COMM_ROWS, D_MODEL), jnp.float32),
            pltpu.VMEM((QBLK, H_LOCAL * DH), jnp.bfloat16),
            pltpu.VMEM((QBLK, H_LOCAL * DH), jnp.bfloat16),
            pltpu.VMEM((SKV, H_LOCAL * DH), jnp.bfloat16),
            pltpu.VMEM((SKV, H_LOCAL * DH), jnp.bfloat16),
            pltpu.SemaphoreType.DMA((3, 3)),
            pltpu.SemaphoreType.DMA((3, 3)),
            pltpu.SemaphoreType.DMA((3, 3)),
            pltpu.SemaphoreType.DMA((3, 3)),
            pltpu.SemaphoreType.REGULAR((3,)),
        ],
        compiler_params=pltpu.CompilerParams(
            dimension_semantics=("arbitrary",),
            has_side_effects=True,
            vmem_limit_bytes=56 * 1024 * 1024,
        ),
    )(
        x[0].astype(jnp.bfloat16),
        Wq.astype(jnp.bfloat16),
        K,
        V,
        Wo.astype(jnp.bfloat16),
    )
    return out[None]
